# baseline (speedup 1.0000x reference)
import functools

import jax
import jax.numpy as jnp
from jax import lax
from jax.experimental import pallas as pl
from jax.experimental.pallas import tpu as pltpu
from jax.experimental.pallas import tpu_sc as plsc

BATCH = 4096
HIST = 200
EMBED_DIM = 64
L = 128


@functools.cache
def _make_gather(V):
  info = plsc.get_sparse_core_info()
  nw = info.num_cores * info.num_subcores
  assert BATCH // L == nw

  mesh = plsc.VectorSubcoreMesh(core_axis_name="c", subcore_axis_name="s")

  @functools.partial(
      pl.kernel,
      mesh=mesh,
      compiler_params=pltpu.CompilerParams(
          use_tc_tiling_on_sc=False, needs_layout_passes=False),
      out_type=jax.ShapeDtypeStruct((HIST, 8, BATCH // L, 8, L), jnp.float32),
      scratch_types=[
          pltpu.VMEM((L,), jnp.int32),
          pltpu.VMEM((L,), jnp.int32),
          pltpu.VMEM((L, EMBED_DIM), jnp.float32),
          pltpu.VMEM((L, EMBED_DIM), jnp.float32),
          pltpu.VMEM((8, 8, L), jnp.float32),
          pltpu.SemaphoreType.DMA,
          pltpu.SemaphoreType.DMA,
      ],
  )
  def gather_kernel(table_hbm, idx_hbm, out_hbm,
                    idx_a, idx_b, rows_a, rows_b, trans_v, sem_a, sem_b):
    wid = lax.axis_index("s") * info.num_cores + lax.axis_index("c")
    col = wid * L
    lane = lax.iota(jnp.int32, 16)

    def fire(h, idx_v, rows_v, sem):
      pltpu.sync_copy(idx_hbm.at[h, pl.ds(col, L)], idx_v)
      pltpu.async_copy(table_hbm.at[idx_v], rows_v, sem)

    def consume(h, idx_v, rows_v, sem):
      pltpu.make_async_copy(table_hbm.at[idx_v], rows_v, sem).wait()

      def tgroup(t1, carry):
        for s in range(8):
          d = t1 * 8 + s
          dcol = lane * 0 + d
          for l0 in range(8):
            v = plsc.load_gather(rows_v, [l0 * 16 + lane, dcol])
            trans_v[t1, s, pl.ds(l0 * 16, 16)] = v
        return carry

      lax.fori_loop(0, 8, tgroup, 0)
      pltpu.sync_copy(trans_v, out_hbm.at[h, :, wid])

    fire(0, idx_a, rows_a, sem_a)

    def body(i, carry):
      h = 2 * i
      fire(h + 1, idx_b, rows_b, sem_b)
      consume(h, idx_a, rows_a, sem_a)
      fire(h + 2, idx_a, rows_a, sem_a)
      consume(h + 1, idx_b, rows_b, sem_b)
      return carry

    lax.fori_loop(0, (HIST - 2) // 2, body, 0)

    fire(HIST - 1, idx_b, rows_b, sem_b)
    consume(HIST - 2, idx_a, rows_a, sem_a)
    consume(HIST - 1, idx_b, rows_b, sem_b)

  return gather_kernel


def kernel(seqTensor, table):
  idx = seqTensor.T.astype(jnp.int32)
  out5 = _make_gather(table.shape[0])(table, idx)
  return out5.transpose(2, 4, 0, 1, 3).reshape(BATCH, HIST, EMBED_DIM)


# speedup vs baseline: 1.0473x; 1.0473x over previous
import functools

import jax
import jax.numpy as jnp
from jax import lax
from jax.experimental import pallas as pl
from jax.experimental.pallas import tpu as pltpu
from jax.experimental.pallas import tpu_sc as plsc

BATCH = 4096
HIST = 200
EMBED_DIM = 64
L = 128


@functools.cache
def _make_gather(V):
  info = plsc.get_sparse_core_info()
  nw = info.num_cores * info.num_subcores
  assert BATCH // L == nw

  mesh = plsc.VectorSubcoreMesh(core_axis_name="c", subcore_axis_name="s")

  @functools.partial(
      pl.kernel,
      mesh=mesh,
      compiler_params=pltpu.CompilerParams(
          use_tc_tiling_on_sc=False, needs_layout_passes=False),
      out_type=jax.ShapeDtypeStruct((HIST, 8, BATCH // L, 1024), jnp.float32),
      scratch_types=[
          pltpu.VMEM((L,), jnp.int32),
          pltpu.VMEM((L,), jnp.int32),
          pltpu.VMEM((L, EMBED_DIM), jnp.float32),
          pltpu.VMEM((L, EMBED_DIM), jnp.float32),
          pltpu.VMEM((8 * 1024,), jnp.float32),
          pltpu.VMEM((8 * 1024,), jnp.float32),
          pltpu.SemaphoreType.DMA,
          pltpu.SemaphoreType.DMA,
          pltpu.SemaphoreType.DMA,
          pltpu.SemaphoreType.DMA,
      ],
  )
  def gather_kernel(table_hbm, idx_hbm, out_hbm,
                    idx_a, idx_b, rows_a, rows_b, tr_a, tr_b,
                    sem_a, sem_b, wsem_a, wsem_b):
    wid = lax.axis_index("s") * info.num_cores + lax.axis_index("c")
    col = wid * L
    lane = lax.iota(jnp.int32, 16)
    ones = lane * 0 + 1
    rowv = [l0 * 16 + lane for l0 in range(8)]

    def fire(h, idx_v, rows_v, sem):
      pltpu.sync_copy(idx_hbm.at[h, pl.ds(col, L)], idx_v)
      pltpu.async_copy(table_hbm.at[idx_v], rows_v, sem)

    def gwait(idx_v, rows_v, sem):
      pltpu.make_async_copy(table_hbm.at[idx_v], rows_v, sem).wait()

    def transpose(rows_v, tr_v):
      def tgroup(t1, dc):
        for s in range(8):
          for l0 in range(8):
            v = plsc.load_gather(rows_v, [rowv[l0], dc])
            tr_v[pl.ds(t1 * 1024 + s * 128 + l0 * 16, 16)] = v
          dc = dc + ones
        return dc
      lax.fori_loop(0, 8, tgroup, lane * 0)

    def wfire(h, tr_v, wsem):
      for t1 in range(8):
        pltpu.async_copy(tr_v.at[pl.ds(t1 * 1024, 1024)],
                         out_hbm.at[h, t1, wid], wsem)

    def wwait(h, tr_v, wsem):
      for t1 in range(8):
        pltpu.make_async_copy(tr_v.at[pl.ds(t1 * 1024, 1024)],
                              out_hbm.at[h, t1, wid], wsem).wait()

    fire(0, idx_a, rows_a, sem_a)

    def body(i, carry):
      h = 2 * i
      fire(h + 1, idx_b, rows_b, sem_b)
      gwait(idx_a, rows_a, sem_a)

      @pl.when(i > 0)
      def _():
        wwait(h, tr_a, wsem_a)

      transpose(rows_a, tr_a)
      wfire(h, tr_a, wsem_a)

      @pl.when(i < HIST // 2 - 1)
      def _():
        fire(h + 2, idx_a, rows_a, sem_a)

      gwait(idx_b, rows_b, sem_b)

      @pl.when(i > 0)
      def _():
        wwait(h, tr_b, wsem_b)

      transpose(rows_b, tr_b)
      wfire(h + 1, tr_b, wsem_b)
      return carry

    lax.fori_loop(0, HIST // 2, body, 0)
    wwait(0, tr_a, wsem_a)
    wwait(0, tr_b, wsem_b)

  return gather_kernel


def kernel(seqTensor, table):
  idx = seqTensor.T.astype(jnp.int32)
  out4 = _make_gather(table.shape[0])(table, idx)
  out5 = out4.reshape(HIST, 8, BATCH // L, 8, L)
  return out5.transpose(2, 4, 0, 1, 3).reshape(BATCH, HIST, EMBED_DIM)


# speedup vs baseline: 1.5100x; 1.4418x over previous
import functools

import jax
import jax.numpy as jnp
from jax import lax
from jax.experimental import pallas as pl
from jax.experimental.pallas import tpu as pltpu
from jax.experimental.pallas import tpu_sc as plsc

BATCH = 4096
HIST = 200
EMBED_DIM = 64
L = 128


@functools.cache
def _make_gather(V):
  info = plsc.get_sparse_core_info()
  nw = info.num_cores * info.num_subcores
  assert BATCH // L == nw

  mesh = plsc.VectorSubcoreMesh(core_axis_name="c", subcore_axis_name="s")

  @functools.partial(
      pl.kernel,
      mesh=mesh,
      compiler_params=pltpu.CompilerParams(
          use_tc_tiling_on_sc=False, needs_layout_passes=False),
      out_type=jax.ShapeDtypeStruct((HIST, 8, BATCH // L, 1024), jnp.float32),
      scratch_types=[
          pltpu.VMEM((L,), jnp.int32),
          pltpu.VMEM((L,), jnp.int32),
          pltpu.VMEM((L, EMBED_DIM), jnp.float32),
          pltpu.VMEM((L, EMBED_DIM), jnp.float32),
          pltpu.VMEM((8 * 1024,), jnp.float32),
          pltpu.VMEM((8 * 1024,), jnp.float32),
          pltpu.SemaphoreType.DMA,
          pltpu.SemaphoreType.DMA,
          pltpu.SemaphoreType.DMA,
          pltpu.SemaphoreType.DMA,
      ],
  )
  def gather_kernel(table_hbm, idx_hbm, out_hbm,
                    idx_a, idx_b, rows_a, rows_b, tr_a, tr_b,
                    sem_a, sem_b, wsem_a, wsem_b):
    wid = lax.axis_index("s") * info.num_cores + lax.axis_index("c")
    col = wid * L
    lane = lax.iota(jnp.int32, 16)
    ones = lane * 0 + 1
    rowv = [l0 * 16 + lane for l0 in range(8)]

    def fire(h, idx_v, rows_v, sem):
      pltpu.sync_copy(idx_hbm.at[h, pl.ds(col, L)], idx_v)
      pltpu.async_copy(table_hbm.at[idx_v], rows_v, sem)

    def gwait(idx_v, rows_v, sem):
      pltpu.make_async_copy(table_hbm.at[idx_v], rows_v, sem).wait()

    def transpose(rows_v, tr_v):
      @plsc.parallel_loop(0, EMBED_DIM, unroll=4, carry=lane * 0)
      def _(d, dc):
        for l0 in range(8):
          v = plsc.load_gather(rows_v, [rowv[l0], dc])
          tr_v[pl.ds(d * 128 + l0 * 16, 16)] = v
        return dc + ones

    def wfire(h, tr_v, wsem):
      for t1 in range(8):
        pltpu.async_copy(tr_v.at[pl.ds(t1 * 1024, 1024)],
                         out_hbm.at[h, t1, wid], wsem)

    def wwait(h, tr_v, wsem):
      for t1 in range(8):
        pltpu.make_async_copy(tr_v.at[pl.ds(t1 * 1024, 1024)],
                              out_hbm.at[h, t1, wid], wsem).wait()

    fire(0, idx_a, rows_a, sem_a)

    def body(i, carry):
      h = 2 * i
      fire(h + 1, idx_b, rows_b, sem_b)
      gwait(idx_a, rows_a, sem_a)

      @pl.when(i > 0)
      def _():
        wwait(h, tr_a, wsem_a)

      transpose(rows_a, tr_a)
      wfire(h, tr_a, wsem_a)

      @pl.when(i < HIST // 2 - 1)
      def _():
        fire(h + 2, idx_a, rows_a, sem_a)

      gwait(idx_b, rows_b, sem_b)

      @pl.when(i > 0)
      def _():
        wwait(h, tr_b, wsem_b)

      transpose(rows_b, tr_b)
      wfire(h + 1, tr_b, wsem_b)
      return carry

    lax.fori_loop(0, HIST // 2, body, 0)
    wwait(0, tr_a, wsem_a)
    wwait(0, tr_b, wsem_b)

  return gather_kernel


def kernel(seqTensor, table):
  idx = seqTensor.T.astype(jnp.int32)
  out4 = _make_gather(table.shape[0])(table, idx)
  out5 = out4.reshape(HIST, 8, BATCH // L, 8, L)
  return out5.transpose(2, 4, 0, 1, 3).reshape(BATCH, HIST, EMBED_DIM)


# speedup vs baseline: 2.4203x; 1.6028x over previous
import functools

import jax
import jax.numpy as jnp
from jax import lax
from jax.experimental import pallas as pl
from jax.experimental.pallas import tpu as pltpu
from jax.experimental.pallas import tpu_sc as plsc

BATCH = 4096
HIST = 200
EMBED_DIM = 64
L = 128


@functools.cache
def _make_gather(V):
  info = plsc.get_sparse_core_info()
  nw = info.num_cores * info.num_subcores
  assert BATCH // L == nw

  mesh = plsc.VectorSubcoreMesh(core_axis_name="c", subcore_axis_name="s")

  @functools.partial(
      pl.kernel,
      mesh=mesh,
      compiler_params=pltpu.CompilerParams(
          use_tc_tiling_on_sc=False, needs_layout_passes=False),
      out_type=jax.ShapeDtypeStruct((HIST, 8, BATCH // L, 8, L), jnp.float32),
      scratch_types=[
          pltpu.VMEM((L,), jnp.int32),
          pltpu.VMEM((L,), jnp.int32),
          pltpu.VMEM((L, EMBED_DIM), jnp.float32),
          pltpu.VMEM((L, EMBED_DIM), jnp.float32),
          pltpu.VMEM((EMBED_DIM, 129), jnp.float32),
          pltpu.VMEM((EMBED_DIM, 129), jnp.float32),
          pltpu.SemaphoreType.DMA,
          pltpu.SemaphoreType.DMA,
          pltpu.SemaphoreType.DMA,
          pltpu.SemaphoreType.DMA,
      ],
  )
  def gather_kernel(table_hbm, idx_hbm, out_hbm,
                    idx_a, idx_b, rows_a, rows_b, tr_a, tr_b,
                    sem_a, sem_b, wsem_a, wsem_b):
    wid = lax.axis_index("s") * info.num_cores + lax.axis_index("c")
    col = wid * L
    lane = lax.iota(jnp.int32, 16)
    ones = lane * 0 + 1
    dvec = [d0 * 16 + lane for d0 in range(4)]

    def fire(h, idx_v, rows_v, sem):
      pltpu.sync_copy(idx_hbm.at[h, pl.ds(col, L)], idx_v)
      pltpu.async_copy(table_hbm.at[idx_v], rows_v, sem)

    def gwait(idx_v, rows_v, sem):
      pltpu.make_async_copy(table_hbm.at[idx_v], rows_v, sem).wait()

    def transpose(rows_v, tr_v):
      @plsc.parallel_loop(0, L, unroll=2, carry=lane * 0)
      def _(tok, tv):
        for d0 in range(4):
          v = rows_v[tok, pl.ds(d0 * 16, 16)]
          plsc.store_scatter(tr_v, [dvec[d0], tv], v)
        return tv + ones

    def wfire(h, tr_v, wsem):
      for t1 in range(8):
        pltpu.async_copy(tr_v.at[pl.ds(t1 * 8, 8), pl.ds(0, L)],
                         out_hbm.at[h, t1, wid], wsem)

    def wwait(h, tr_v, wsem):
      for t1 in range(8):
        pltpu.make_async_copy(tr_v.at[pl.ds(t1 * 8, 8), pl.ds(0, L)],
                              out_hbm.at[h, t1, wid], wsem).wait()

    fire(0, idx_a, rows_a, sem_a)

    def body(i, carry):
      h = 2 * i
      fire(h + 1, idx_b, rows_b, sem_b)
      gwait(idx_a, rows_a, sem_a)

      @pl.when(i > 0)
      def _():
        wwait(h, tr_a, wsem_a)

      transpose(rows_a, tr_a)
      wfire(h, tr_a, wsem_a)

      @pl.when(i < HIST // 2 - 1)
      def _():
        fire(h + 2, idx_a, rows_a, sem_a)

      gwait(idx_b, rows_b, sem_b)

      @pl.when(i > 0)
      def _():
        wwait(h, tr_b, wsem_b)

      transpose(rows_b, tr_b)
      wfire(h + 1, tr_b, wsem_b)
      return carry

    lax.fori_loop(0, HIST // 2, body, 0)
    wwait(0, tr_a, wsem_a)
    wwait(0, tr_b, wsem_b)

  return gather_kernel


def kernel(seqTensor, table):
  idx = seqTensor.T.astype(jnp.int32)
  out5 = _make_gather(table.shape[0])(table, idx)
  return out5.transpose(2, 4, 0, 1, 3).reshape(BATCH, HIST, EMBED_DIM)


# speedup vs baseline: 2.5732x; 1.0632x over previous
import functools

import jax
import jax.numpy as jnp
from jax import lax
from jax.experimental import pallas as pl
from jax.experimental.pallas import tpu as pltpu
from jax.experimental.pallas import tpu_sc as plsc

BATCH = 4096
HIST = 200
EMBED_DIM = 64
L = 128


@functools.cache
def _make_gather(V):
  info = plsc.get_sparse_core_info()
  nw = info.num_cores * info.num_subcores
  assert BATCH // L == nw

  mesh = plsc.VectorSubcoreMesh(core_axis_name="c", subcore_axis_name="s")

  @functools.partial(
      pl.kernel,
      mesh=mesh,
      compiler_params=pltpu.CompilerParams(
          use_tc_tiling_on_sc=False, needs_layout_passes=False),
      out_type=jax.ShapeDtypeStruct((HIST, 8, BATCH // L, 8, L), jnp.float32),
      scratch_types=[
          pltpu.VMEM((HIST, L), jnp.int32),
          pltpu.VMEM((L, EMBED_DIM), jnp.float32),
          pltpu.VMEM((L, EMBED_DIM), jnp.float32),
          pltpu.VMEM((EMBED_DIM, 129), jnp.float32),
          pltpu.VMEM((EMBED_DIM, 129), jnp.float32),
          pltpu.SemaphoreType.DMA,
          pltpu.SemaphoreType.DMA,
          pltpu.SemaphoreType.DMA,
          pltpu.SemaphoreType.DMA,
      ],
  )
  def gather_kernel(table_hbm, idx_hbm, out_hbm,
                    idx_all, rows_a, rows_b, tr_a, tr_b,
                    sem_a, sem_b, wsem_a, wsem_b):
    wid = lax.axis_index("s") * info.num_cores + lax.axis_index("c")
    col = wid * L
    lane = lax.iota(jnp.int32, 16)
    ones = lane * 0 + 1
    dvec = [d0 * 16 + lane for d0 in range(4)]

    pltpu.sync_copy(idx_hbm.at[:, pl.ds(col, L)], idx_all)

    def fire(h, rows_v, sem):
      pltpu.async_copy(table_hbm.at[idx_all.at[h]], rows_v, sem)

    def gwait(h, rows_v, sem):
      pltpu.make_async_copy(table_hbm.at[idx_all.at[h]], rows_v, sem).wait()

    def transpose(rows_v, tr_v):
      @plsc.parallel_loop(0, L, unroll=2, carry=lane * 0)
      def _(tok, tv):
        for d0 in range(4):
          v = rows_v[tok, pl.ds(d0 * 16, 16)]
          plsc.store_scatter(tr_v, [dvec[d0], tv], v)
        return tv + ones

    def wfire(h, tr_v, wsem):
      for t1 in range(8):
        pltpu.async_copy(tr_v.at[pl.ds(t1 * 8, 8), pl.ds(0, L)],
                         out_hbm.at[h, t1, wid], wsem)

    def wwait(h, tr_v, wsem):
      for t1 in range(8):
        pltpu.make_async_copy(tr_v.at[pl.ds(t1 * 8, 8), pl.ds(0, L)],
                              out_hbm.at[h, t1, wid], wsem).wait()

    fire(0, rows_a, sem_a)

    def body(i, carry):
      h = 2 * i
      fire(h + 1, rows_b, sem_b)
      gwait(h, rows_a, sem_a)

      @pl.when(i > 0)
      def _():
        wwait(h, tr_a, wsem_a)

      transpose(rows_a, tr_a)
      wfire(h, tr_a, wsem_a)

      @pl.when(i < HIST // 2 - 1)
      def _():
        fire(h + 2, rows_a, sem_a)

      gwait(h + 1, rows_b, sem_b)

      @pl.when(i > 0)
      def _():
        wwait(h, tr_b, wsem_b)

      transpose(rows_b, tr_b)
      wfire(h + 1, tr_b, wsem_b)
      return carry

    lax.fori_loop(0, HIST // 2, body, 0)
    wwait(0, tr_a, wsem_a)
    wwait(0, tr_b, wsem_b)

  return gather_kernel


def kernel(seqTensor, table):
  idx = seqTensor.T.astype(jnp.int32)
  out5 = _make_gather(table.shape[0])(table, idx)
  return out5.transpose(2, 4, 0, 1, 3).reshape(BATCH, HIST, EMBED_DIM)


# speedup vs baseline: 2.5780x; 1.0019x over previous
import functools

import jax
import jax.numpy as jnp
from jax import lax
from jax.experimental import pallas as pl
from jax.experimental.pallas import tpu as pltpu
from jax.experimental.pallas import tpu_sc as plsc

BATCH = 4096
HIST = 200
EMBED_DIM = 64
L = 128


@functools.cache
def _make_gather(V):
  info = plsc.get_sparse_core_info()
  nw = info.num_cores * info.num_subcores
  assert BATCH // L == nw

  mesh = plsc.VectorSubcoreMesh(core_axis_name="c", subcore_axis_name="s")

  @functools.partial(
      pl.kernel,
      mesh=mesh,
      compiler_params=pltpu.CompilerParams(
          use_tc_tiling_on_sc=False, needs_layout_passes=False),
      out_type=jax.ShapeDtypeStruct((HIST, 8, BATCH // L, 8, L), jnp.float32),
      scratch_types=[
          pltpu.VMEM((HIST, L), jnp.int32),
          pltpu.VMEM((L, EMBED_DIM), jnp.float32),
          pltpu.VMEM((L, EMBED_DIM), jnp.float32),
          pltpu.VMEM((EMBED_DIM, 129), jnp.float32),
          pltpu.VMEM((EMBED_DIM, 129), jnp.float32),
          pltpu.SemaphoreType.DMA,
          pltpu.SemaphoreType.DMA,
          pltpu.SemaphoreType.DMA,
          pltpu.SemaphoreType.DMA,
      ],
  )
  def gather_kernel(table_hbm, idx_hbm, out_hbm,
                    idx_all, rows_a, rows_b, tr_a, tr_b,
                    sem_a, sem_b, wsem_a, wsem_b):
    wid = lax.axis_index("s") * info.num_cores + lax.axis_index("c")
    col = wid * L
    lane = lax.iota(jnp.int32, 16)
    ones = lane * 0 + 1
    dvec = [d0 * 16 + lane for d0 in range(4)]

    pltpu.sync_copy(idx_hbm.at[:, pl.ds(col, L)], idx_all)

    def fire(h, rows_v, sem):
      pltpu.async_copy(table_hbm.at[idx_all.at[h]], rows_v, sem)

    def gwait(h, rows_v, sem):
      pltpu.make_async_copy(table_hbm.at[idx_all.at[h]], rows_v, sem).wait()

    def transpose(rows_v, tr_v):
      @plsc.parallel_loop(0, L, unroll=4, carry=lane * 0)
      def _(tok, tv):
        for d0 in range(4):
          v = rows_v[tok, pl.ds(d0 * 16, 16)]
          plsc.store_scatter(tr_v, [dvec[d0], tv], v)
        return tv + ones

    def wfire(h, tr_v, wsem):
      for t1 in range(8):
        pltpu.async_copy(tr_v.at[pl.ds(t1 * 8, 8), pl.ds(0, L)],
                         out_hbm.at[h, t1, wid], wsem)

    def wwait(h, tr_v, wsem):
      for t1 in range(8):
        pltpu.make_async_copy(tr_v.at[pl.ds(t1 * 8, 8), pl.ds(0, L)],
                              out_hbm.at[h, t1, wid], wsem).wait()

    fire(0, rows_a, sem_a)

    def body(i, carry):
      h = 2 * i
      fire(h + 1, rows_b, sem_b)
      gwait(h, rows_a, sem_a)

      @pl.when(i > 0)
      def _():
        wwait(h, tr_a, wsem_a)

      transpose(rows_a, tr_a)
      wfire(h, tr_a, wsem_a)

      @pl.when(i < HIST // 2 - 1)
      def _():
        fire(h + 2, rows_a, sem_a)

      gwait(h + 1, rows_b, sem_b)

      @pl.when(i > 0)
      def _():
        wwait(h, tr_b, wsem_b)

      transpose(rows_b, tr_b)
      wfire(h + 1, tr_b, wsem_b)
      return carry

    lax.fori_loop(0, HIST // 2, body, 0)
    wwait(0, tr_a, wsem_a)
    wwait(0, tr_b, wsem_b)

  return gather_kernel


def kernel(seqTensor, table):
  idx = seqTensor.T.astype(jnp.int32)
  out5 = _make_gather(table.shape[0])(table, idx)
  return out5.transpose(2, 4, 0, 1, 3).reshape(BATCH, HIST, EMBED_DIM)
